# Initial kernel scaffold; baseline (speedup 1.0000x reference)
import jax
import jax.numpy as jnp
from jax.experimental import pallas as pl


def kernel(geometry_points, surface_points, volume_points, geo_tokens, surf_tokens, vol_tokens, params):
    raise NotImplementedError("write your pallas kernel here")


# speedup vs baseline: 11.7929x; 11.7929x over previous
import functools

import jax
import jax.numpy as jnp
from jax import lax
from jax.experimental import pallas as pl
from jax.experimental.pallas import tpu as pltpu
from jax.experimental.pallas import tpu_sc as plsc

_R1, _R2 = 0.1, 0.2
_K1, _K2 = 16, 32
_KTOT = _K1 + _K2
_N = 2048
_QT = 256
_BQH = 64
_HID = 1024

_NC = 2
_NS = 16
_NW = _NC * _NS
_QPW = _N // _NW
_GRP = 8
_CHUNKS = _N // 16


def _enc_body(q_ref, st_ref, enc_ref):
    q = q_ref[...]
    s = st_ref[...]
    dx = q[:, 0:1] - s[0:1, :]
    dy = q[:, 1:2] - s[1:2, :]
    dz = q[:, 2:3] - s[2:3, :]
    d2 = (dx * dx + dy * dy) + dz * dz
    enc = (d2 <= _R2 * _R2).astype(jnp.float32) + (d2 <= _R1 * _R1).astype(
        jnp.float32)
    enc_ref[...] = enc


def _enc(qpad, st):
    return pl.pallas_call(
        _enc_body,
        grid=(_N // _QT,),
        in_specs=[
            pl.BlockSpec((_QT, 8), lambda i: (i, 0)),
            pl.BlockSpec((8, _N), lambda i: (0, 0)),
        ],
        out_specs=pl.BlockSpec((_QT, _N), lambda i: (i, 0)),
        out_shape=jax.ShapeDtypeStruct((_N, _N), jnp.float32),
    )(qpad, st)


def _sc_body(enc_hbm, table_hbm, out_hbm, table_v, rowbuf, stage, i1, i2):
    wid = lax.axis_index("s") * _NC + lax.axis_index("c")
    base_q = wid * _QPW
    pltpu.sync_copy(table_hbm, table_v)
    iota16 = lax.iota(jnp.int32, 16)
    zc = jnp.zeros((16,), jnp.int32)

    def group(g, _):
        pltpu.sync_copy(enc_hbm.at[pl.ds(base_q + g * _GRP, _GRP)], rowbuf)
        for qq in range(_GRP):
            qloc = g * _GRP + qq

            def chunk(c, carry):
                cnt1, cnt2 = carry
                v = rowbuf[qq, pl.ds(c * 16, 16)]
                m2 = v >= 0.5
                m1 = v >= 1.5
                ids = c * 16 + iota16
                pos1 = cnt1 + plsc.cumsum(m1.astype(jnp.int32)) - 1
                pos2 = cnt2 + plsc.cumsum(m2.astype(jnp.int32)) - 1
                plsc.store_scatter(i1, [pos1], ids, mask=m1 & (pos1 < _K1))
                plsc.store_scatter(i2, [pos2], ids, mask=m2 & (pos2 < _K2))
                return (cnt1 + plsc.all_reduce_population_count(m1),
                        cnt2 + plsc.all_reduce_population_count(m2))

            cnt1, cnt2 = lax.fori_loop(0, _CHUNKS, chunk, (zc, zc))
            qv = jnp.full((16,), qloc, jnp.int32)

            def emit(slot_base, idx_raw, valid):
                idxs = jnp.where(valid, idx_raw, 0)
                slots = slot_base + iota16
                for cc in range(3):
                    col = jnp.full((16,), cc, jnp.int32)
                    gv = plsc.load_gather(table_v, [idxs, col])
                    plsc.store_scatter(stage, [slots, qv, col], gv)
                plsc.store_scatter(stage, [slots, qv, jnp.full((16,), 3, jnp.int32)],
                                   valid.astype(jnp.float32))

            c1 = jnp.minimum(cnt1, _K1)
            emit(0, i1[...], iota16 < c1)
            c2 = jnp.minimum(cnt2, _K2)
            emit(_K1, i2[pl.ds(0, 16)], iota16 < c2)
            emit(_K1 + 16, i2[pl.ds(16, 16)], (16 + iota16) < c2)
        return 0

    lax.fori_loop(0, _QPW // _GRP, group, 0)
    pltpu.sync_copy(stage, out_hbm.at[:, pl.ds(base_q, _QPW), :])


def _sc_select_gather(enc, table):
    mesh = plsc.VectorSubcoreMesh(core_axis_name="c", subcore_axis_name="s")
    f = functools.partial(
        pl.kernel,
        out_type=jax.ShapeDtypeStruct((_KTOT, _N, 8), jnp.float32),
        mesh=mesh,
        scratch_types=[
            pltpu.VMEM((_N, 4), jnp.float32),
            pltpu.VMEM((_GRP, _N), jnp.float32),
            pltpu.VMEM((_KTOT, _QPW, 8), jnp.float32),
            pltpu.VMEM((_K1,), jnp.int32),
            pltpu.VMEM((_K2,), jnp.int32),
        ],
        compiler_params=pltpu.CompilerParams(needs_layout_passes=False, use_tc_tiling_on_sc=False),
    )(_sc_body)
    return f(enc, table)


def _mlp_body(coords_ref, q_ref, w1a_ref, b1a_ref, w2a_ref, b2a_ref,
              w1b_ref, b1b_ref, w2b_ref, b2b_ref, wo_ref, bo_ref,
              base_ref, out_ref):
    q = q_ref[...]
    colmask = (lax.broadcasted_iota(jnp.int32, (1, 8), 1) < 3).astype(
        jnp.float32)

    def scale_feats(kbase, kk, w1, b1, w2, b2):
        acc = jnp.zeros((_QT, _BQH), jnp.float32)
        for k in range(kk):
            c = coords_ref[kbase + k]
            valid = c[:, 3:4]
            rel = (c - q) * colmask
            h = jnp.maximum(
                jnp.dot(rel, w1, preferred_element_type=jnp.float32) + b1, 0.0)
            h = jnp.maximum(
                jnp.dot(h, w2, preferred_element_type=jnp.float32) + b2, 0.0)
            acc = jnp.maximum(acc, h * valid)
        return acc

    f1 = scale_feats(0, _K1, w1a_ref[...], b1a_ref[...], w2a_ref[...],
                     b2a_ref[...])
    f2 = scale_feats(_K1, _K2, w1b_ref[...], b1b_ref[...], w2b_ref[...],
                     b2b_ref[...])
    wo = wo_ref[...]
    out_ref[...] = (base_ref[...]
                    + jnp.dot(f1, wo[:_BQH], preferred_element_type=jnp.float32)
                    + jnp.dot(f2, wo[_BQH:], preferred_element_type=jnp.float32)
                    + bo_ref[...])


def _mlp(coords, qpad, p, base):
    sa, sb = p["scales"]
    w1a = jnp.pad(sa["W1"], ((0, 5), (0, 0)))
    w1b = jnp.pad(sb["W1"], ((0, 5), (0, 0)))
    full = lambda i: (0, 0)
    return pl.pallas_call(
        _mlp_body,
        grid=(_N // _QT,),
        in_specs=[
            pl.BlockSpec((_KTOT, _QT, 8), lambda i: (0, i, 0)),
            pl.BlockSpec((_QT, 8), lambda i: (i, 0)),
            pl.BlockSpec((8, _BQH), full),
            pl.BlockSpec((1, _BQH), full),
            pl.BlockSpec((_BQH, _BQH), full),
            pl.BlockSpec((1, _BQH), full),
            pl.BlockSpec((8, _BQH), full),
            pl.BlockSpec((1, _BQH), full),
            pl.BlockSpec((_BQH, _BQH), full),
            pl.BlockSpec((1, _BQH), full),
            pl.BlockSpec((2 * _BQH, _HID), full),
            pl.BlockSpec((1, _HID), full),
            pl.BlockSpec((_QT, _HID), lambda i: (i, 0)),
        ],
        out_specs=pl.BlockSpec((_QT, _HID), lambda i: (i, 0)),
        out_shape=jax.ShapeDtypeStruct((_N, _HID), jnp.float32),
    )(coords, qpad, w1a, sa["b1"][None], sa["W2"], sa["b2"][None],
      w1b, sb["b1"][None], sb["W2"], sb["b2"][None],
      p["Wout"], p["bout"][None], base)


def _stack(qprep, sprep, p, base):
    qpad, _, _ = qprep
    _, st, table = sprep
    enc = _enc(qpad, st)
    coords = _sc_select_gather(enc, table)
    return _mlp(coords, qpad, p, base)


def kernel(geometry_points, surface_points, volume_points, geo_tokens,
           surf_tokens, vol_tokens, params):
    def prep(pts):
        p = pts[0]
        qpad = jnp.pad(p, ((0, 0), (0, 5)))
        st = qpad.T
        table = jnp.pad(p, ((0, 0), (0, 1)))
        return qpad, st, table

    g = prep(geometry_points)
    s = prep(surface_points)
    v = prep(volume_points)

    geo = _stack(g, g, params["geo_self"], geo_tokens[0])
    geo = _stack(g, s, params["geo_surf"], geo)
    surf = _stack(s, s, params["surf_self"], surf_tokens[0])
    surf = _stack(s, g, params["surf_geo"], surf)
    geo = _stack(g, v, params["geo_vol"], geo)
    vol = _stack(v, v, params["vol_self"], vol_tokens[0])
    vol = _stack(v, g, params["vol_geo"], vol)
    return (geo[None], surf[None], vol[None])
